# Initial kernel scaffold; baseline (speedup 1.0000x reference)
import jax
import jax.numpy as jnp
from jax.experimental import pallas as pl


def kernel(x, edge_index, graph_indices):
    raise NotImplementedError("write your pallas kernel here")


# speedup vs baseline: 5.3356x; 5.3356x over previous
import jax
import jax.numpy as jnp
from jax import lax
from jax.experimental import pallas as pl
from jax.experimental.pallas import tpu as pltpu
from jax.experimental.pallas import tpu_sc as plsc

N = 10000
D = 256
G = 64
LANES = 16
NJ = D // LANES
CH = 64

NC = 2
NS = 16
NW = NC * NS
SEGS_PER_W = G // NW
SEARCH_STEPS = 14


def _pool_body(x_hbm, gi_hbm, out_hbm, gi_v, buf, stage):
    wid = lax.axis_index("s") * NC + lax.axis_index("c")
    _worker(wid, x_hbm, gi_hbm, out_hbm, gi_v, buf, stage)


def _worker(wid, x_hbm, gi_hbm, out_hbm, gi_v, buf, stage):
    pltpu.sync_copy(gi_hbm, gi_v.at[pl.ds(0, N)])
    gi_v[pl.ds(N, LANES)] = jnp.full((LANES,), G, jnp.int32)

    def lower_bound(g):
        def step(_, lh):
            lo, hi = lh
            mid = (lo + hi) // 2
            p = gi_v[pl.ds(mid, LANES)][0] < g
            return (jnp.where(p, mid + 1, lo), jnp.where(p, hi, mid))
        lo, _ = lax.fori_loop(0, SEARCH_STEPS, step,
                              (jnp.int32(0), jnp.int32(N)))
        return lo

    g0 = wid * SEGS_PER_W
    bounds = [lower_bound(g0 + t) for t in range(SEGS_PER_W + 1)]

    for t in range(SEGS_PER_W):
        g = g0 + t
        s, e = bounds[t], bounds[t + 1]
        cnt = e - s
        nch = (cnt + (CH - 1)) // CH

        def chunk_body(c, acc, s=s, e=e):
            rs = s + c * CH
            base = jnp.minimum(rs, N - CH)
            off = rs - base
            lim = off + jnp.minimum(jnp.int32(CH), e - rs)
            pltpu.sync_copy(x_hbm.at[pl.ds(base * D, CH * D)], buf)

            def row_body(r, acc2):
                sums, maxs = acc2
                new_s, new_m = [], []
                for j in range(NJ):
                    v = buf[pl.ds(r * D + j * LANES, LANES)]
                    new_s.append(sums[j] + v)
                    new_m.append(jnp.maximum(maxs[j], v))
                return (tuple(new_s), tuple(new_m))

            return lax.fori_loop(off, lim, row_body, acc)

        zero = jnp.zeros((LANES,), jnp.float32)
        ninf = jnp.full((LANES,), -jnp.inf, jnp.float32)
        init = (tuple(zero for _ in range(NJ)),
                tuple(ninf for _ in range(NJ)))
        sums, maxs = lax.fori_loop(0, nch, chunk_body, init)

        cntf = jnp.maximum(cnt, 1).astype(jnp.float32)
        inv = jnp.full((LANES,), 1.0, jnp.float32) \
            / jnp.full((LANES,), cntf, jnp.float32)
        for j in range(NJ):
            stage[pl.ds(j * LANES, LANES)] = maxs[j]
            stage[pl.ds(D + j * LANES, LANES)] = sums[j] * inv
            stage[pl.ds(2 * D + j * LANES, LANES)] = sums[j]
        pltpu.sync_copy(stage, out_hbm.at[pl.ds(g * 3 * D, 3 * D)])


def kernel(x, edge_index, graph_indices):
    del edge_index
    mesh = plsc.VectorSubcoreMesh(core_axis_name="c", subcore_axis_name="s",
                                  num_cores=NC, num_subcores=NS)
    f = pl.kernel(
        _pool_body,
        out_type=jax.ShapeDtypeStruct((G * 3 * D,), jnp.float32),
        mesh=mesh,
        scratch_types=[
            pltpu.VMEM((N + LANES,), jnp.int32),
            pltpu.VMEM((CH * D,), jnp.float32),
            pltpu.VMEM((3 * D,), jnp.float32),
        ],
    )
    return f(x.reshape(-1), graph_indices).reshape(G, 3 * D)
